# Initial kernel scaffold; baseline (speedup 1.0000x reference)
import jax
import jax.numpy as jnp
from jax.experimental import pallas as pl


def kernel(idx, table, ln_weight, ln_bias):
    raise NotImplementedError("write your pallas kernel here")


# speedup vs baseline: 1.0826x; 1.0826x over previous
import functools

import jax
import jax.numpy as jnp
from jax import lax
from jax.experimental import pallas as pl
from jax.experimental.pallas import tpu as pltpu
from jax.experimental.pallas import tpu_sc as plsc

DIM = 64
EPS = 1e-5
NC = 2
NS = 16
NW = NC * NS
IG = 128
CHUNK = 1024
GPC = CHUNK // IG


def _ln_chunk(rows, w_regs, b_regs):
    inv_d = 1.0 / DIM

    def body(r, carry):
        v0 = rows[r, pl.ds(0, 16)]
        v1 = rows[r, pl.ds(16, 16)]
        v2 = rows[r, pl.ds(32, 16)]
        v3 = rows[r, pl.ds(48, 16)]
        s = (v0 + v1) + (v2 + v3)
        q = (v0 * v0 + v1 * v1) + (v2 * v2 + v3 * v3)
        t = jnp.sum(s)
        u = jnp.sum(q)
        mean = t * inv_d
        var = u * inv_d - mean * mean
        x = var + EPS
        i = lax.bitcast_convert_type(x, jnp.int32)
        i = jnp.int32(0x5F3759DF) - lax.shift_right_logical(i, 1)
        y = lax.bitcast_convert_type(i, jnp.float32)
        hx = x * 0.5
        y = y * (1.5 - hx * y * y)
        y = y * (1.5 - hx * y * y)
        y = y * (1.5 - hx * y * y)
        c = -(mean * y)
        vs = (v0, v1, v2, v3)
        for k in range(4):
            n = vs[k] * y + c
            rows[r, pl.ds(16 * k, 16)] = n * w_regs[k] + b_regs[k]
        return carry

    lax.fori_loop(0, CHUNK, body, 0)


def _make_sc_kernel(b_total):
    b_per_w = b_total // NW
    idx_groups = b_per_w // IG
    n_chunk = b_per_w // CHUNK

    def body(table, idxg, w_hbm, bias_hbm, out, idx_v, rows, wb_v, gsem):
        wid = lax.axis_index("s") * NC + lax.axis_index("c")
        row0 = wid * b_per_w
        pltpu.sync_copy(idxg.at[pl.ds(wid * idx_groups, idx_groups)], idx_v)
        pltpu.sync_copy(w_hbm, wb_v.at[0])
        pltpu.sync_copy(bias_hbm, wb_v.at[1])
        w_regs = [wb_v[0, pl.ds(16 * k, 16)] for k in range(4)]
        b_regs = [wb_v[1, pl.ds(16 * k, 16)] for k in range(4)]

        def chunk_body(c, carry):
            copies = [
                pltpu.async_copy(
                    table.at[idx_v.at[c * GPC + j]],
                    rows.at[pl.ds(j * IG, IG)], gsem)
                for j in range(GPC)
            ]
            for cp in copies:
                cp.wait()
            _ln_chunk(rows, w_regs, b_regs)
            pltpu.sync_copy(rows, out.at[pl.ds(row0 + c * CHUNK, CHUNK)])
            return carry

        lax.fori_loop(0, n_chunk, chunk_body, 0)

    return pl.kernel(
        body,
        out_type=jax.ShapeDtypeStruct((b_total, DIM), jnp.float32),
        mesh=plsc.VectorSubcoreMesh(core_axis_name="c", subcore_axis_name="s"),
        compiler_params=pltpu.CompilerParams(
            needs_layout_passes=False, use_tc_tiling_on_sc=False),
        scratch_types=[
            pltpu.VMEM((idx_groups, IG), jnp.int32),
            pltpu.VMEM((CHUNK, DIM), jnp.float32),
            pltpu.VMEM((2, DIM), jnp.float32),
            pltpu.SemaphoreType.DMA,
        ],
    )


def kernel(idx, table, ln_weight, ln_bias):
    b, l = idx.shape
    b_total = b * l
    idxg = idx.reshape(b_total // IG, IG).astype(jnp.int32)
    out = _make_sc_kernel(b_total)(table, idxg, ln_weight, ln_bias)
    return out.reshape(b, l, DIM)


# speedup vs baseline: 1.1499x; 1.0621x over previous
import functools

import jax
import jax.numpy as jnp
from jax import lax
from jax.experimental import pallas as pl
from jax.experimental.pallas import tpu as pltpu
from jax.experimental.pallas import tpu_sc as plsc

DIM = 64
EPS = 1e-5
NC = 2
NS = 16
NW = NC * NS
IG = 128
CHUNK = 1024
GPC = CHUNK // IG


def _ln_chunk(rows, w_regs, b_regs):
    inv_d = 1.0 / DIM

    def body(r, carry):
        v0 = rows[r, pl.ds(0, 16)]
        v1 = rows[r, pl.ds(16, 16)]
        v2 = rows[r, pl.ds(32, 16)]
        v3 = rows[r, pl.ds(48, 16)]
        s = (v0 + v1) + (v2 + v3)
        q = (v0 * v0 + v1 * v1) + (v2 * v2 + v3 * v3)
        t = jnp.sum(s)
        u = jnp.sum(q)
        mean = t * inv_d
        var = u * inv_d - mean * mean
        x = var + EPS
        i = lax.bitcast_convert_type(x, jnp.int32)
        i = jnp.int32(0x5F3759DF) - lax.shift_right_logical(i, 1)
        y = lax.bitcast_convert_type(i, jnp.float32)
        hx = x * 0.5
        y = y * (1.5 - hx * y * y)
        y = y * (1.5 - hx * y * y)
        c = -(mean * y)
        vs = (v0, v1, v2, v3)
        for k in range(4):
            n = vs[k] * y + c
            rows[r, pl.ds(16 * k, 16)] = n * w_regs[k] + b_regs[k]
        return carry

    lax.fori_loop(0, CHUNK, body, 0, unroll=8)


def _make_sc_kernel(b_total):
    b_per_w = b_total // NW
    idx_groups = b_per_w // IG
    n_chunk = b_per_w // CHUNK

    def body(table, idxg, w_hbm, bias_hbm, out, idx_v, rows, wb_v, gsem):
        wid = lax.axis_index("s") * NC + lax.axis_index("c")
        row0 = wid * b_per_w
        pltpu.sync_copy(idxg.at[pl.ds(wid * idx_groups, idx_groups)], idx_v)
        pltpu.sync_copy(w_hbm, wb_v.at[0])
        pltpu.sync_copy(bias_hbm, wb_v.at[1])
        w_regs = [wb_v[0, pl.ds(16 * k, 16)] for k in range(4)]
        b_regs = [wb_v[1, pl.ds(16 * k, 16)] for k in range(4)]

        def chunk_body(c, carry):
            copies = [
                pltpu.async_copy(
                    table.at[idx_v.at[c * GPC + j]],
                    rows.at[pl.ds(j * IG, IG)], gsem)
                for j in range(GPC)
            ]
            for cp in copies:
                cp.wait()
            _ln_chunk(rows, w_regs, b_regs)
            pltpu.sync_copy(rows, out.at[pl.ds(row0 + c * CHUNK, CHUNK)])
            return carry

        lax.fori_loop(0, n_chunk, chunk_body, 0)

    return pl.kernel(
        body,
        out_type=jax.ShapeDtypeStruct((b_total, DIM), jnp.float32),
        mesh=plsc.VectorSubcoreMesh(core_axis_name="c", subcore_axis_name="s"),
        compiler_params=pltpu.CompilerParams(
            needs_layout_passes=False, use_tc_tiling_on_sc=False),
        scratch_types=[
            pltpu.VMEM((idx_groups, IG), jnp.int32),
            pltpu.VMEM((CHUNK, DIM), jnp.float32),
            pltpu.VMEM((2, DIM), jnp.float32),
            pltpu.SemaphoreType.DMA,
        ],
    )


def kernel(idx, table, ln_weight, ln_bias):
    b, l = idx.shape
    b_total = b * l
    idxg = idx.reshape(b_total // IG, IG).astype(jnp.int32)
    out = _make_sc_kernel(b_total)(table, idxg, ln_weight, ln_bias)
    return out.reshape(b, l, DIM)


# speedup vs baseline: 1.4002x; 1.2177x over previous
import functools

import jax
import jax.numpy as jnp
from jax import lax
from jax.experimental import pallas as pl
from jax.experimental.pallas import tpu as pltpu
from jax.experimental.pallas import tpu_sc as plsc

DIM = 64
EPS = 1e-5
NC = 2
NS = 16
NW = NC * NS
IG = 128
CHUNK = 1024
GPC = CHUNK // IG


def _ln_chunk(rows, tbuf, qbuf, abuf, cbuf, w_regs, b_regs, mask15):
    inv_d = 1.0 / DIM

    def pass1(r, carry):
        v0 = rows[r, pl.ds(0, 16)]
        v1 = rows[r, pl.ds(16, 16)]
        v2 = rows[r, pl.ds(32, 16)]
        v3 = rows[r, pl.ds(48, 16)]
        s = (v0 + v1) + (v2 + v3)
        q = (v0 * v0 + v1 * v1) + (v2 * v2 + v3 * v3)
        rsplat = jnp.full((16,), r, jnp.int32)
        plsc.store_scatter(tbuf, [rsplat], plsc.cumsum(s), mask=mask15)
        plsc.store_scatter(qbuf, [rsplat], plsc.cumsum(q), mask=mask15)
        return carry

    lax.fori_loop(0, CHUNK, pass1, 0, unroll=8)

    def pass2(g, carry):
        off = pl.multiple_of(g * 16, 16)
        t = tbuf[pl.ds(off, 16)]
        u = qbuf[pl.ds(off, 16)]
        mean = t * inv_d
        var = u * inv_d - mean * mean
        x = var + EPS
        i = plsc.bitcast(x, jnp.int32)
        i = jnp.int32(0x5F3759DF) - lax.shift_right_logical(i, 1)
        y = plsc.bitcast(i, jnp.float32)
        hx = x * 0.5
        y = y * (1.5 - hx * y * y)
        y = y * (1.5 - hx * y * y)
        abuf[pl.ds(off, 16)] = y
        cbuf[pl.ds(off, 16)] = -(mean * y)
        return carry

    lax.fori_loop(0, CHUNK // 16, pass2, 0, unroll=2)

    def pass3(r, carry):
        rsplat = jnp.full((16,), r, jnp.int32)
        a = plsc.load_gather(abuf, [rsplat])
        c = plsc.load_gather(cbuf, [rsplat])
        for k in range(4):
            n = rows[r, pl.ds(16 * k, 16)] * a + c
            rows[r, pl.ds(16 * k, 16)] = n * w_regs[k] + b_regs[k]
        return carry

    lax.fori_loop(0, CHUNK, pass3, 0, unroll=8)


def _make_sc_kernel(b_total):
    b_per_w = b_total // NW
    idx_groups = b_per_w // IG
    n_chunk = b_per_w // CHUNK

    def body(table, idxg, w_hbm, bias_hbm, out, idx_v, rows, tbuf, qbuf,
             abuf, cbuf, wb_v, gsem):
        wid = lax.axis_index("s") * NC + lax.axis_index("c")
        row0 = wid * b_per_w
        pltpu.sync_copy(idxg.at[pl.ds(wid * idx_groups, idx_groups)], idx_v)
        pltpu.sync_copy(w_hbm, wb_v.at[0])
        pltpu.sync_copy(bias_hbm, wb_v.at[1])
        w_regs = [wb_v[0, pl.ds(16 * k, 16)] for k in range(4)]
        b_regs = [wb_v[1, pl.ds(16 * k, 16)] for k in range(4)]
        mask15 = lax.iota(jnp.int32, 16) == 15

        def chunk_body(c, carry):
            copies = [
                pltpu.async_copy(
                    table.at[idx_v.at[c * GPC + j]],
                    rows.at[pl.ds(j * IG, IG)], gsem)
                for j in range(GPC)
            ]
            for cp in copies:
                cp.wait()
            _ln_chunk(rows, tbuf, qbuf, abuf, cbuf, w_regs, b_regs, mask15)
            pltpu.sync_copy(rows, out.at[pl.ds(row0 + c * CHUNK, CHUNK)])
            return carry

        lax.fori_loop(0, n_chunk, chunk_body, 0)

    return pl.kernel(
        body,
        out_type=jax.ShapeDtypeStruct((b_total, DIM), jnp.float32),
        mesh=plsc.VectorSubcoreMesh(core_axis_name="c", subcore_axis_name="s"),
        compiler_params=pltpu.CompilerParams(
            needs_layout_passes=False, use_tc_tiling_on_sc=False),
        scratch_types=[
            pltpu.VMEM((idx_groups, IG), jnp.int32),
            pltpu.VMEM((CHUNK, DIM), jnp.float32),
            pltpu.VMEM((CHUNK,), jnp.float32),
            pltpu.VMEM((CHUNK,), jnp.float32),
            pltpu.VMEM((CHUNK,), jnp.float32),
            pltpu.VMEM((CHUNK,), jnp.float32),
            pltpu.VMEM((2, DIM), jnp.float32),
            pltpu.SemaphoreType.DMA,
        ],
    )


def kernel(idx, table, ln_weight, ln_bias):
    b, l = idx.shape
    b_total = b * l
    idxg = idx.reshape(b_total // IG, IG).astype(jnp.int32)
    out = _make_sc_kernel(b_total)(table, idxg, ln_weight, ln_bias)
    return out.reshape(b, l, DIM)


# speedup vs baseline: 1.5211x; 1.0863x over previous
import functools

import jax
import jax.numpy as jnp
from jax import lax
from jax.experimental import pallas as pl
from jax.experimental.pallas import tpu as pltpu
from jax.experimental.pallas import tpu_sc as plsc

DIM = 64
EPS = 1e-5
NC = 2
NS = 16
NW = NC * NS
IG = 128
CHUNK = 256
GPC = CHUNK // IG
NBUF = 4


def _ln_chunk(rows, tbuf, qbuf, abuf, cbuf, w_regs, b_regs, mask15):
    inv_d = 1.0 / DIM

    def pass1(r, carry):
        v0 = rows[r, pl.ds(0, 16)]
        v1 = rows[r, pl.ds(16, 16)]
        v2 = rows[r, pl.ds(32, 16)]
        v3 = rows[r, pl.ds(48, 16)]
        s = (v0 + v1) + (v2 + v3)
        q = (v0 * v0 + v1 * v1) + (v2 * v2 + v3 * v3)
        rsplat = jnp.full((16,), r, jnp.int32)
        plsc.store_scatter(tbuf, [rsplat], plsc.cumsum(s), mask=mask15)
        plsc.store_scatter(qbuf, [rsplat], plsc.cumsum(q), mask=mask15)
        return carry

    lax.fori_loop(0, CHUNK, pass1, 0, unroll=8)

    def pass2(g, carry):
        off = pl.multiple_of(g * 16, 16)
        t = tbuf[pl.ds(off, 16)]
        u = qbuf[pl.ds(off, 16)]
        mean = t * inv_d
        var = u * inv_d - mean * mean
        x = var + EPS
        i = plsc.bitcast(x, jnp.int32)
        i = jnp.int32(0x5F3759DF) - lax.shift_right_logical(i, 1)
        y = plsc.bitcast(i, jnp.float32)
        hx = x * 0.5
        y = y * (1.5 - hx * y * y)
        y = y * (1.5 - hx * y * y)
        abuf[pl.ds(off, 16)] = y
        cbuf[pl.ds(off, 16)] = -(mean * y)
        return carry

    lax.fori_loop(0, CHUNK // 16, pass2, 0, unroll=2)

    def pass3(r, carry):
        rsplat = jnp.full((16,), r, jnp.int32)
        a = plsc.load_gather(abuf, [rsplat])
        c = plsc.load_gather(cbuf, [rsplat])
        for k in range(4):
            n = rows[r, pl.ds(16 * k, 16)] * a + c
            rows[r, pl.ds(16 * k, 16)] = n * w_regs[k] + b_regs[k]
        return carry

    lax.fori_loop(0, CHUNK, pass3, 0, unroll=8)


def _make_sc_kernel(b_total):
    b_per_w = b_total // NW
    idx_groups = b_per_w // IG
    n_chunk = b_per_w // CHUNK
    n_outer = n_chunk // NBUF

    def body(table, idxg, w_hbm, bias_hbm, out, idx_v, r0, r1, r2, r3,
             tbuf, qbuf, abuf, cbuf, wb_v,
             g0, g1, g2, g3, w0, w1, w2, w3):
        bufs = (r0, r1, r2, r3)
        gsems = (g0, g1, g2, g3)
        wsems = (w0, w1, w2, w3)
        wid = lax.axis_index("s") * NC + lax.axis_index("c")
        row0 = wid * b_per_w
        pltpu.sync_copy(idxg.at[pl.ds(wid * idx_groups, idx_groups)], idx_v)
        pltpu.sync_copy(w_hbm, wb_v.at[0])
        pltpu.sync_copy(bias_hbm, wb_v.at[1])
        w_regs = [wb_v[0, pl.ds(16 * k, 16)] for k in range(4)]
        b_regs = [wb_v[1, pl.ds(16 * k, 16)] for k in range(4)]
        mask15 = lax.iota(jnp.int32, 16) == 15

        def issue_gather(c, buf, sem):
            for j in range(GPC):
                pltpu.async_copy(table.at[idx_v.at[c * GPC + j]],
                                 buf.at[pl.ds(j * IG, IG)], sem)

        def wait_gather(buf, sem):
            pltpu.make_async_copy(out.at[pl.ds(0, CHUNK)], buf, sem).wait()

        def issue_wb(c, buf, sem):
            pltpu.async_copy(buf, out.at[pl.ds(row0 + c * CHUNK, CHUNK)], sem)

        def wait_wb(buf, sem):
            pltpu.make_async_copy(buf, out.at[pl.ds(0, CHUNK)], sem).wait()

        issue_gather(0, bufs[0], gsems[0])
        issue_gather(1, bufs[1], gsems[1])

        def outer(i, carry):
            for k in range(NBUF):
                c = i * NBUF + k
                b = k
                wait_gather(bufs[b], gsems[b])
                _ln_chunk(bufs[b], tbuf, qbuf, abuf, cbuf,
                          w_regs, b_regs, mask15)
                issue_wb(c, bufs[b], wsems[b])
                b2 = (k + 2) % NBUF

                @pl.when(jnp.logical_and(c + 2 < n_chunk, c >= 2))
                def _():
                    wait_wb(bufs[b2], wsems[b2])

                @pl.when(c + 2 < n_chunk)
                def _():
                    issue_gather(c + 2, bufs[b2], gsems[b2])

            return carry

        lax.fori_loop(0, n_outer, outer, 0)
        wait_wb(bufs[(n_chunk - 2) % NBUF], wsems[(n_chunk - 2) % NBUF])
        wait_wb(bufs[(n_chunk - 1) % NBUF], wsems[(n_chunk - 1) % NBUF])

    return pl.kernel(
        body,
        out_type=jax.ShapeDtypeStruct((b_total, DIM), jnp.float32),
        mesh=plsc.VectorSubcoreMesh(core_axis_name="c", subcore_axis_name="s"),
        compiler_params=pltpu.CompilerParams(
            needs_layout_passes=False, use_tc_tiling_on_sc=False),
        scratch_types=[
            pltpu.VMEM((idx_groups, IG), jnp.int32),
            pltpu.VMEM((CHUNK, DIM), jnp.float32),
            pltpu.VMEM((CHUNK, DIM), jnp.float32),
            pltpu.VMEM((CHUNK, DIM), jnp.float32),
            pltpu.VMEM((CHUNK, DIM), jnp.float32),
            pltpu.VMEM((CHUNK,), jnp.float32),
            pltpu.VMEM((CHUNK,), jnp.float32),
            pltpu.VMEM((CHUNK,), jnp.float32),
            pltpu.VMEM((CHUNK,), jnp.float32),
            pltpu.VMEM((2, DIM), jnp.float32),
            pltpu.SemaphoreType.DMA,
            pltpu.SemaphoreType.DMA,
            pltpu.SemaphoreType.DMA,
            pltpu.SemaphoreType.DMA,
            pltpu.SemaphoreType.DMA,
            pltpu.SemaphoreType.DMA,
            pltpu.SemaphoreType.DMA,
            pltpu.SemaphoreType.DMA,
        ],
    )


def kernel(idx, table, ln_weight, ln_bias):
    b, l = idx.shape
    b_total = b * l
    idxg = idx.reshape(b_total // IG, IG).astype(jnp.int32)
    out = _make_sc_kernel(b_total)(table, idxg, ln_weight, ln_bias)
    return out.reshape(b, l, DIM)
